# baseline (speedup 1.0000x reference)
import functools

import jax
import jax.numpy as jnp
from jax.experimental import pallas as pl
from jax.experimental.pallas import tpu as pltpu
from jax.experimental.pallas import tpu_sc as plsc

EPS = 1e-5
B = 8
N = 2048
K = 32
M = B * N * K
RB = 256
KT = 4
MB = KT * N
GW = 128
TPB = K // KT


def _knn_body(Cc, xc_ref, out_ref):
    full = xc_ref[0]
    cols = xc_ref[0, :, pl.ds(pl.program_id(1) * RB, RB)]
    if Cc <= 16:
        CR = 3
        x2m = (full[0] * full[0] + full[1] * full[1]) + full[2] * full[2]
        x2n = (cols[0] * cols[0] + cols[1] * cols[1]) + cols[2] * cols[2]
        fb = full.astype(jnp.bfloat16).astype(jnp.float32)
        cb = cols.astype(jnp.bfloat16).astype(jnp.float32)
        inner = (cb[0][:, None] * fb[0][None, :]
                 + cb[1][:, None] * fb[1][None, :]) \
            + cb[2][:, None] * fb[2][None, :]
    else:
        x2m = jnp.sum(full * full, axis=0)
        x2n = jnp.sum(cols * cols, axis=0)
        inner = jax.lax.dot_general(
            cols.astype(jnp.bfloat16), full.astype(jnp.bfloat16),
            (((0,), (0,)), ((), ())),
            preferred_element_type=jnp.float32)
    scores = 2.0 * inner - x2n[:, None] - x2m[None, :]
    lane = jax.lax.broadcasted_iota(jnp.int32, (RB, N), 1)
    base = pl.program_id(0) * N
    for j in range(K):
        mx = jnp.max(scores, axis=1, keepdims=True)
        cand = jnp.where(scores == mx, lane, N)
        sel = jnp.min(cand, axis=1)
        out_ref[0, j, :] = sel + base
        scores = jnp.where(lane == sel[:, None], -jnp.inf, scores)


def _knn_indices(xc, Cc):
    return pl.pallas_call(
        functools.partial(_knn_body, Cc),
        grid=(B, N // RB),
        in_specs=[
            pl.BlockSpec((1, Cc, N), lambda b, i: (b, 0, 0)),
        ],
        out_specs=pl.BlockSpec((1, K, RB), lambda b, i: (b, 0, i)),
        out_shape=jax.ShapeDtypeStruct((B, K, N), jnp.int32),
        compiler_params=pltpu.CompilerParams(
            dimension_semantics=("parallel", "parallel")),
    )(xc)


def _sc_gather(src, idx):
    mesh = plsc.VectorSubcoreMesh(core_axis_name="c", subcore_axis_name="s")
    win = 128

    @functools.partial(
        pl.kernel,
        out_type=jax.ShapeDtypeStruct((M, GW), src.dtype),
        mesh=mesh,
    )
    def gather_kernel(src_hbm, i_hbm, o_hbm):
        def body(i_vmem, o_vmem):
            pltpu.sync_copy(src_hbm.at[i_vmem.at[0]], o_vmem)

        pltpu.emit_pipeline(
            body,
            grid=(M // win,),
            in_specs=[pl.BlockSpec((1, win), lambda i: (0, i))],
            out_specs=[pl.BlockSpec((win, GW), lambda i: (i, 0))],
            core_axis_name=("c", "s"),
            dimension_semantics=(pltpu.PARALLEL,),
        )(i_hbm, o_hbm)

    return gather_kernel(src, idx)


def _lrelu(z):
    return jnp.where(z > 0, z, 0.2 * z)


def _y1_body(xn_ref, ct_ref, w1t_ref, o_ref):
    ce = jnp.reshape(jnp.broadcast_to(ct_ref[...][None], (KT, N, GW)),
                     (MB, GW))
    delta = (xn_ref[...] - ce).astype(jnp.bfloat16)
    g = jnp.concatenate([delta, ce.astype(jnp.bfloat16)], axis=1)
    y = jnp.dot(g, w1t_ref[...], preferred_element_type=jnp.float32)
    for kk in range(KT):
        o_ref[0, kk] = y[kk * N:(kk + 1) * N, :].T


def _bn_lrelu3(y, m_ref, d_ref, g_ref, b_ref):
    z = (y - m_ref[...][:, :, None]) / d_ref[...][:, :, None]
    z = z * g_ref[...][:, :, None] + b_ref[...][:, :, None]
    return _lrelu(z)


def _y2_body(y1_ref, m1_ref, d1_ref, g1_ref, b1_ref, w2_ref, o_ref):
    h1 = _bn_lrelu3(y1_ref[0], m1_ref, d1_ref, g1_ref, b1_ref)
    h1 = h1.astype(jnp.bfloat16)
    for kk in range(KT):
        o_ref[0, kk] = jax.lax.dot_general(
            w2_ref[...], h1[kk], (((1,), (0,)), ((), ())),
            preferred_element_type=jnp.float32)


def _pool_body(y2_ref, m2_ref, d2_ref, g2_ref, b2_ref, xt_out_ref, ocm_ref):
    h2 = _bn_lrelu3(y2_ref[0], m2_ref, d2_ref, g2_ref, b2_ref)
    part = jnp.max(h2, axis=0)
    first = pl.program_id(0) % TPB == 0

    @pl.when(first)
    def _():
        ocm_ref[0] = part

    @pl.when(jnp.logical_not(first))
    def _():
        ocm_ref[0] = jnp.maximum(ocm_ref[0], part)

    last = pl.program_id(0) % TPB == TPB - 1

    @pl.when(last)
    def _():
        o = ocm_ref[0].T
        xt_out_ref[:, 0:64] = o
        xt_out_ref[:, 64:GW] = jnp.zeros((N, GW - 64), jnp.float32)


def _edge_layer(xt_flat, xc, w1, g1, b1, w2, g2, b2, C):
    pad = ((0, 0), (0, GW - C))
    w1t = jnp.concatenate([jnp.pad(w1[:, :C], pad).T,
                           jnp.pad(w1[:, C:], pad).T]).astype(jnp.bfloat16)
    w2b = w2.astype(jnp.bfloat16)

    gidx = _knn_indices(xc, xc.shape[1])
    xn = _sc_gather(xt_flat, gidx.reshape(1, M))

    grid = (M // MB,)
    xn_spec = pl.BlockSpec((MB, GW), lambda i: (i, 0))
    ct_spec = pl.BlockSpec((N, GW), lambda i: (i // TPB, 0))
    w1_spec = pl.BlockSpec((2 * GW, 64), lambda i: (0, 0))
    w2_spec = pl.BlockSpec((64, 64), lambda i: (0, 0))
    c_spec = pl.BlockSpec((1, 64), lambda i: (0, 0))
    y_spec = pl.BlockSpec((1, KT, 64, N), lambda i: (i // TPB, i % TPB, 0, 0))
    y_shape = jax.ShapeDtypeStruct((B, K, 64, N), jnp.float32)

    y1 = pl.pallas_call(
        _y1_body,
        grid=grid,
        in_specs=[xn_spec, ct_spec, w1_spec],
        out_specs=y_spec,
        out_shape=y_shape,
        compiler_params=pltpu.CompilerParams(
            dimension_semantics=("parallel",)),
    )(xn, xt_flat, w1t)
    y1t = jnp.transpose(y1, (0, 2, 3, 1))
    m1 = jnp.mean(y1t, axis=(0, 2, 3))
    d1 = jnp.sqrt(jnp.var(y1t, axis=(0, 2, 3)) + EPS)

    y2 = pl.pallas_call(
        _y2_body,
        grid=grid,
        in_specs=[y_spec] + [c_spec] * 4 + [w2_spec],
        out_specs=y_spec,
        out_shape=y_shape,
        compiler_params=pltpu.CompilerParams(
            dimension_semantics=("parallel",)),
    )(y1, m1[None], d1[None], g1[None], b1[None], w2b)
    y2t = jnp.transpose(y2, (0, 2, 3, 1))
    m2 = jnp.mean(y2t, axis=(0, 2, 3))
    d2 = jnp.sqrt(jnp.var(y2t, axis=(0, 2, 3)) + EPS)

    xt_next, ocm = pl.pallas_call(
        _pool_body,
        grid=grid,
        in_specs=[y_spec] + [c_spec] * 4,
        out_specs=[
            pl.BlockSpec((N, GW), lambda i: (i // TPB, 0)),
            pl.BlockSpec((1, 64, N), lambda i: (i // TPB, 0, 0)),
        ],
        out_shape=[
            jax.ShapeDtypeStruct((B * N, GW), jnp.float32),
            jax.ShapeDtypeStruct((B, 64, N), jnp.float32),
        ],
    )(y2, m2[None], d2[None], g2[None], b2[None])
    return xt_next, ocm


def kernel(x, coordinate, w1_0, gamma1_0, beta1_0, w2_0, gamma2_0, beta2_0, w1_1, gamma1_1, beta1_1, w2_1, gamma2_1, beta2_1, w1_2, gamma1_2, beta1_2, w2_2, gamma2_2, beta2_2):
    xc0 = jnp.pad(x, ((0, 0), (0, 13), (0, 0)))
    xt0 = jnp.pad(jnp.transpose(x, (0, 2, 1)).reshape(B * N, 3),
                  ((0, 0), (0, GW - 3)))
    xt1, o0 = _edge_layer(xt0, xc0, w1_0, gamma1_0, beta1_0,
                          w2_0, gamma2_0, beta2_0, C=3)
    xt2, o1 = _edge_layer(xt1, o0, w1_1, gamma1_1, beta1_1,
                          w2_1, gamma2_1, beta2_1, C=64)
    _, o2 = _edge_layer(xt2, o1, w1_2, gamma1_2, beta1_2,
                        w2_2, gamma2_2, beta2_2, C=64)
    cat = jnp.concatenate([o0, o1, o2], axis=1)
    return (cat, o0, o1, o2)


# speedup vs baseline: 1.0381x; 1.0381x over previous
import functools

import jax
import jax.numpy as jnp
from jax.experimental import pallas as pl
from jax.experimental.pallas import tpu as pltpu
from jax.experimental.pallas import tpu_sc as plsc

EPS = 1e-5
B = 8
N = 2048
K = 32
M = B * N * K
RB = 256
KT = 4
MB = KT * N
GW = 128
TPB = K // KT


def _knn_body(Cc, xc_ref, out_ref):
    full = xc_ref[0]
    cols = xc_ref[0, :, pl.ds(pl.program_id(1) * RB, RB)]
    if Cc <= 16:
        CR = 3
        x2m = (full[0] * full[0] + full[1] * full[1]) + full[2] * full[2]
        x2n = (cols[0] * cols[0] + cols[1] * cols[1]) + cols[2] * cols[2]
        fb = full.astype(jnp.bfloat16).astype(jnp.float32)
        cb = cols.astype(jnp.bfloat16).astype(jnp.float32)
        inner = (cb[0][:, None] * fb[0][None, :]
                 + cb[1][:, None] * fb[1][None, :]) \
            + cb[2][:, None] * fb[2][None, :]
    else:
        x2m = jnp.sum(full * full, axis=0)
        x2n = jnp.sum(cols * cols, axis=0)
        inner = jax.lax.dot_general(
            cols.astype(jnp.bfloat16), full.astype(jnp.bfloat16),
            (((0,), (0,)), ((), ())),
            preferred_element_type=jnp.float32)
    scores = 2.0 * inner - x2n[:, None] - x2m[None, :]
    lane = jax.lax.broadcasted_iota(jnp.int32, (RB, N), 1)
    base = pl.program_id(0) * N
    for j in range(K):
        mx = jnp.max(scores, axis=1, keepdims=True)
        cand = jnp.where(scores == mx, lane, N)
        sel = jnp.min(cand, axis=1)
        out_ref[0, j, :] = sel + base
        scores = jnp.where(lane == sel[:, None], -jnp.inf, scores)


def _knn_indices(xc, Cc):
    return pl.pallas_call(
        functools.partial(_knn_body, Cc),
        grid=(B, N // RB),
        in_specs=[
            pl.BlockSpec((1, Cc, N), lambda b, i: (b, 0, 0)),
        ],
        out_specs=pl.BlockSpec((1, K, RB), lambda b, i: (b, 0, i)),
        out_shape=jax.ShapeDtypeStruct((B, K, N), jnp.int32),
        compiler_params=pltpu.CompilerParams(
            dimension_semantics=("parallel", "parallel")),
    )(xc)


def _sc_gather(src, idx):
    mesh = plsc.VectorSubcoreMesh(core_axis_name="c", subcore_axis_name="s")
    win = 128

    @functools.partial(
        pl.kernel,
        out_type=jax.ShapeDtypeStruct((M, GW), src.dtype),
        mesh=mesh,
    )
    def gather_kernel(src_hbm, i_hbm, o_hbm):
        def body(i_vmem, o_vmem):
            pltpu.sync_copy(src_hbm.at[i_vmem.at[0]], o_vmem)

        pltpu.emit_pipeline(
            body,
            grid=(M // win,),
            in_specs=[pl.BlockSpec((1, win), lambda i: (0, i))],
            out_specs=[pl.BlockSpec((win, GW), lambda i: (i, 0))],
            core_axis_name=("c", "s"),
            dimension_semantics=(pltpu.PARALLEL,),
        )(i_hbm, o_hbm)

    return gather_kernel(src, idx)


def _lrelu(z):
    return jnp.where(z > 0, z, 0.2 * z)


def _y1_body(xn_ref, ct_ref, w1t_ref, o_ref):
    ce = jnp.reshape(jnp.broadcast_to(ct_ref[...][None], (KT, N, GW)),
                     (MB, GW))
    delta = (xn_ref[...] - ce).astype(jnp.bfloat16)
    g = jnp.concatenate([delta, ce.astype(jnp.bfloat16)], axis=1)
    y = jnp.dot(g, w1t_ref[...], preferred_element_type=jnp.float32)
    for kk in range(KT):
        o_ref[0, kk] = y[kk * N:(kk + 1) * N, :].T


def _bn_lrelu3(y, m_ref, d_ref, g_ref, b_ref):
    z = (y - m_ref[...][:, :, None]) / d_ref[...][:, :, None]
    z = z * g_ref[...][:, :, None] + b_ref[...][:, :, None]
    return _lrelu(z)


def _y2_body(y1_ref, m1_ref, d1_ref, g1_ref, b1_ref, w2_ref, o_ref):
    h1 = _bn_lrelu3(y1_ref[0], m1_ref, d1_ref, g1_ref, b1_ref)
    h1 = h1.astype(jnp.bfloat16)
    for kk in range(KT):
        o_ref[0, kk] = jax.lax.dot_general(
            w2_ref[...], h1[kk], (((1,), (0,)), ((), ())),
            preferred_element_type=jnp.float32)


def _pool_body(y2_ref, m2_ref, d2_ref, g2_ref, b2_ref, xt_out_ref, ocm_ref):
    h2 = _bn_lrelu3(y2_ref[0], m2_ref, d2_ref, g2_ref, b2_ref)
    part = jnp.max(h2, axis=0)
    first = pl.program_id(0) % TPB == 0

    @pl.when(first)
    def _():
        ocm_ref[0] = part

    @pl.when(jnp.logical_not(first))
    def _():
        ocm_ref[0] = jnp.maximum(ocm_ref[0], part)

    last = pl.program_id(0) % TPB == TPB - 1

    @pl.when(last)
    def _():
        o = ocm_ref[0].T
        xt_out_ref[:, 0:64] = o
        xt_out_ref[:, 64:GW] = jnp.zeros((N, GW - 64), jnp.float32)


def _edge_layer(xt_flat, xc, w1, g1, b1, w2, g2, b2, C):
    pad = ((0, 0), (0, GW - C))
    w1t = jnp.concatenate([jnp.pad(w1[:, :C], pad).T,
                           jnp.pad(w1[:, C:], pad).T]).astype(jnp.bfloat16)
    w2b = w2.astype(jnp.bfloat16)

    gidx = _knn_indices(xc, xc.shape[1])
    xn = _sc_gather(xt_flat, gidx.reshape(1, M))

    grid = (M // MB,)
    xn_spec = pl.BlockSpec((MB, GW), lambda i: (i, 0))
    ct_spec = pl.BlockSpec((N, GW), lambda i: (i // TPB, 0))
    w1_spec = pl.BlockSpec((2 * GW, 64), lambda i: (0, 0))
    w2_spec = pl.BlockSpec((64, 64), lambda i: (0, 0))
    c_spec = pl.BlockSpec((1, 64), lambda i: (0, 0))
    y_spec = pl.BlockSpec((1, KT, 64, N), lambda i: (i // TPB, i % TPB, 0, 0))
    y_shape = jax.ShapeDtypeStruct((B, K, 64, N), jnp.float32)

    y1 = pl.pallas_call(
        _y1_body,
        grid=grid,
        in_specs=[xn_spec, ct_spec, w1_spec],
        out_specs=y_spec,
        out_shape=y_shape,
        compiler_params=pltpu.CompilerParams(
            dimension_semantics=("parallel",)),
    )(xn, xt_flat, w1t)
    m1 = jnp.mean(y1, axis=(0, 1, 3))
    d1 = jnp.sqrt(jnp.var(y1, axis=(0, 1, 3)) + EPS)

    y2 = pl.pallas_call(
        _y2_body,
        grid=grid,
        in_specs=[y_spec] + [c_spec] * 4 + [w2_spec],
        out_specs=y_spec,
        out_shape=y_shape,
        compiler_params=pltpu.CompilerParams(
            dimension_semantics=("parallel",)),
    )(y1, m1[None], d1[None], g1[None], b1[None], w2b)
    m2 = jnp.mean(y2, axis=(0, 1, 3))
    d2 = jnp.sqrt(jnp.var(y2, axis=(0, 1, 3)) + EPS)

    xt_next, ocm = pl.pallas_call(
        _pool_body,
        grid=grid,
        in_specs=[y_spec] + [c_spec] * 4,
        out_specs=[
            pl.BlockSpec((N, GW), lambda i: (i // TPB, 0)),
            pl.BlockSpec((1, 64, N), lambda i: (i // TPB, 0, 0)),
        ],
        out_shape=[
            jax.ShapeDtypeStruct((B * N, GW), jnp.float32),
            jax.ShapeDtypeStruct((B, 64, N), jnp.float32),
        ],
    )(y2, m2[None], d2[None], g2[None], b2[None])
    return xt_next, ocm


def kernel(x, coordinate, w1_0, gamma1_0, beta1_0, w2_0, gamma2_0, beta2_0, w1_1, gamma1_1, beta1_1, w2_1, gamma2_1, beta2_1, w1_2, gamma1_2, beta1_2, w2_2, gamma2_2, beta2_2):
    xc0 = jnp.pad(x, ((0, 0), (0, 13), (0, 0)))
    xt0 = jnp.pad(jnp.transpose(x, (0, 2, 1)).reshape(B * N, 3),
                  ((0, 0), (0, GW - 3)))
    xt1, o0 = _edge_layer(xt0, xc0, w1_0, gamma1_0, beta1_0,
                          w2_0, gamma2_0, beta2_0, C=3)
    xt2, o1 = _edge_layer(xt1, o0, w1_1, gamma1_1, beta1_1,
                          w2_1, gamma2_1, beta2_1, C=64)
    _, o2 = _edge_layer(xt2, o1, w1_2, gamma1_2, beta1_2,
                        w2_2, gamma2_2, beta2_2, C=64)
    cat = jnp.concatenate([o0, o1, o2], axis=1)
    return (cat, o0, o1, o2)


# speedup vs baseline: 1.0383x; 1.0002x over previous
import functools

import jax
import jax.numpy as jnp
from jax.experimental import pallas as pl
from jax.experimental.pallas import tpu as pltpu
from jax.experimental.pallas import tpu_sc as plsc

EPS = 1e-5
B = 8
N = 2048
K = 32
M = B * N * K
RB = 256
KT = 4
MB = KT * N
GW = 128
TPB = K // KT


def _knn_body(Cc, xc_ref, out_ref):
    full = xc_ref[0]
    cols = xc_ref[0, :, pl.ds(pl.program_id(1) * RB, RB)]
    if Cc <= 16:
        x2m = (full[0] * full[0] + full[1] * full[1]) + full[2] * full[2]
        x2n = (cols[0] * cols[0] + cols[1] * cols[1]) + cols[2] * cols[2]
        fb = full.astype(jnp.bfloat16).astype(jnp.float32)
        cb = cols.astype(jnp.bfloat16).astype(jnp.float32)
        inner = (cb[0][:, None] * fb[0][None, :]
                 + cb[1][:, None] * fb[1][None, :]) \
            + cb[2][:, None] * fb[2][None, :]
    else:
        x2m = jnp.sum(full * full, axis=0)
        x2n = jnp.sum(cols * cols, axis=0)
        inner = jax.lax.dot_general(
            cols.astype(jnp.bfloat16), full.astype(jnp.bfloat16),
            (((0,), (0,)), ((), ())),
            preferred_element_type=jnp.float32)
    scores = 2.0 * inner - x2n[:, None] - x2m[None, :]
    lane = jax.lax.broadcasted_iota(jnp.int32, (RB, N), 1)
    base = pl.program_id(0) * N
    for j in range(K):
        mx = jnp.max(scores, axis=1, keepdims=True)
        cand = jnp.where(scores == mx, lane, N)
        sel = jnp.min(cand, axis=1)
        out_ref[0, j, :] = sel + base
        scores = jnp.where(lane == sel[:, None], -jnp.inf, scores)


def _knn_indices(xc, Cc):
    return pl.pallas_call(
        functools.partial(_knn_body, Cc),
        grid=(B, N // RB),
        in_specs=[
            pl.BlockSpec((1, Cc, N), lambda b, i: (b, 0, 0)),
        ],
        out_specs=pl.BlockSpec((1, K, RB), lambda b, i: (b, 0, i)),
        out_shape=jax.ShapeDtypeStruct((B, K, N), jnp.int32),
        compiler_params=pltpu.CompilerParams(
            dimension_semantics=("parallel", "parallel")),
    )(xc)


def _sc_gather(src, idx):
    mesh = plsc.VectorSubcoreMesh(core_axis_name="c", subcore_axis_name="s")
    win = 128

    @functools.partial(
        pl.kernel,
        out_type=jax.ShapeDtypeStruct((M, GW), src.dtype),
        mesh=mesh,
    )
    def gather_kernel(src_hbm, i_hbm, o_hbm):
        def body(i_vmem, o_vmem):
            pltpu.sync_copy(src_hbm.at[i_vmem.at[0]], o_vmem)

        pltpu.emit_pipeline(
            body,
            grid=(M // win,),
            in_specs=[pl.BlockSpec((1, win), lambda i: (0, i))],
            out_specs=[pl.BlockSpec((win, GW), lambda i: (i, 0))],
            core_axis_name=("c", "s"),
            dimension_semantics=(pltpu.PARALLEL,),
        )(i_hbm, o_hbm)

    return gather_kernel(src, idx)


def _lrelu(z):
    return jnp.where(z > 0, z, 0.2 * z)


def _y1_body(xn_ref, ct_ref, w1t_ref, o_ref):
    ce = jnp.reshape(jnp.broadcast_to(ct_ref[...][None], (KT, N, GW)),
                     (MB, GW))
    delta = (xn_ref[...] - ce).astype(jnp.bfloat16)
    g = jnp.concatenate([delta, ce.astype(jnp.bfloat16)], axis=1)
    y = jnp.dot(g, w1t_ref[...], preferred_element_type=jnp.float32)
    for kk in range(KT):
        o_ref[0, kk] = y[kk * N:(kk + 1) * N, :].T


def _bn_lrelu3(y, m_ref, d_ref, g_ref, b_ref):
    z = (y - m_ref[...][:, :, None]) / d_ref[...][:, :, None]
    z = z * g_ref[...][:, :, None] + b_ref[...][:, :, None]
    return _lrelu(z)


def _y2_body(y1_ref, m1_ref, d1_ref, g1_ref, b1_ref, w2_ref, o_ref):
    h1 = _bn_lrelu3(y1_ref[0], m1_ref, d1_ref, g1_ref, b1_ref)
    h1 = h1.astype(jnp.bfloat16)
    for kk in range(KT):
        o_ref[0, kk] = jax.lax.dot_general(
            w2_ref[...], h1[kk], (((1,), (0,)), ((), ())),
            preferred_element_type=jnp.float32)


def _pool_body(y2_ref, m2_ref, d2_ref, g2_ref, b2_ref, xt_out_ref, ocm_ref):
    h2 = _bn_lrelu3(y2_ref[0], m2_ref, d2_ref, g2_ref, b2_ref)
    part = jnp.max(h2, axis=0)
    first = pl.program_id(0) % TPB == 0

    @pl.when(first)
    def _():
        ocm_ref[0] = part

    @pl.when(jnp.logical_not(first))
    def _():
        ocm_ref[0] = jnp.maximum(ocm_ref[0], part)

    last = pl.program_id(0) % TPB == TPB - 1

    @pl.when(last)
    def _():
        o = ocm_ref[0].T
        xt_out_ref[:, 0:64] = o
        xt_out_ref[:, 64:GW] = jnp.zeros((N, GW - 64), jnp.float32)


def _edge_layer(xt_flat, xc, w1, g1, b1, w2, g2, b2, C):
    pad = ((0, 0), (0, GW - C))
    w1t = jnp.concatenate([jnp.pad(w1[:, :C], pad).T,
                           jnp.pad(w1[:, C:], pad).T]).astype(jnp.bfloat16)
    w2b = w2.astype(jnp.bfloat16)

    gidx = _knn_indices(xc, xc.shape[1])
    xn = _sc_gather(xt_flat, gidx.reshape(1, M))

    grid = (M // MB,)
    xn_spec = pl.BlockSpec((MB, GW), lambda i: (i, 0))
    ct_spec = pl.BlockSpec((N, GW), lambda i: (i // TPB, 0))
    w1_spec = pl.BlockSpec((2 * GW, 64), lambda i: (0, 0))
    w2_spec = pl.BlockSpec((64, 64), lambda i: (0, 0))
    c_spec = pl.BlockSpec((1, 64), lambda i: (0, 0))
    y_spec = pl.BlockSpec((1, KT, 64, N), lambda i: (i // TPB, i % TPB, 0, 0))
    y_shape = jax.ShapeDtypeStruct((B, K, 64, N), jnp.float32)

    y1 = pl.pallas_call(
        _y1_body,
        grid=grid,
        in_specs=[xn_spec, ct_spec, w1_spec],
        out_specs=y_spec,
        out_shape=y_shape,
        compiler_params=pltpu.CompilerParams(
            dimension_semantics=("parallel",)),
    )(xn, xt_flat, w1t)
    m1 = jnp.mean(y1, axis=(0, 1, 3))
    d1 = jnp.sqrt(jnp.var(y1, axis=(0, 1, 3)) + EPS)

    y2 = pl.pallas_call(
        _y2_body,
        grid=grid,
        in_specs=[y_spec] + [c_spec] * 4 + [w2_spec],
        out_specs=y_spec,
        out_shape=y_shape,
        compiler_params=pltpu.CompilerParams(
            dimension_semantics=("parallel",)),
    )(y1, m1[None], d1[None], g1[None], b1[None], w2b)
    m2 = jnp.mean(y2, axis=(0, 1, 3))
    d2 = jnp.sqrt(jnp.var(y2, axis=(0, 1, 3)) + EPS)

    xt_next, ocm = pl.pallas_call(
        _pool_body,
        grid=grid,
        in_specs=[y_spec] + [c_spec] * 4,
        out_specs=[
            pl.BlockSpec((N, GW), lambda i: (i // TPB, 0)),
            pl.BlockSpec((1, 64, N), lambda i: (i // TPB, 0, 0)),
        ],
        out_shape=[
            jax.ShapeDtypeStruct((B * N, GW), jnp.float32),
            jax.ShapeDtypeStruct((B, 64, N), jnp.float32),
        ],
    )(y2, m2[None], d2[None], g2[None], b2[None])
    return xt_next, ocm


def kernel(x, coordinate, w1_0, gamma1_0, beta1_0, w2_0, gamma2_0, beta2_0, w1_1, gamma1_1, beta1_1, w2_1, gamma2_1, beta2_1, w1_2, gamma1_2, beta1_2, w2_2, gamma2_2, beta2_2):
    xc0 = jnp.pad(x, ((0, 0), (0, 13), (0, 0)))
    xt0 = jnp.pad(jnp.transpose(x, (0, 2, 1)).reshape(B * N, 3),
                  ((0, 0), (0, GW - 3)))
    xt1, o0 = _edge_layer(xt0, xc0, w1_0, gamma1_0, beta1_0,
                          w2_0, gamma2_0, beta2_0, C=3)
    xt2, o1 = _edge_layer(xt1, o0, w1_1, gamma1_1, beta1_1,
                          w2_1, gamma2_1, beta2_1, C=64)
    _, o2 = _edge_layer(xt2, o1, w1_2, gamma1_2, beta1_2,
                        w2_2, gamma2_2, beta2_2, C=64)
    cat = jnp.concatenate([o0, o1, o2], axis=1)
    return (cat, o0, o1, o2)
